# Initial kernel scaffold; baseline (speedup 1.0000x reference)
import jax
import jax.numpy as jnp
from jax.experimental import pallas as pl


def kernel(seq_len, table):
    raise NotImplementedError("write your pallas kernel here")


# speedup vs baseline: 4.2175x; 4.2175x over previous
import functools

import jax
import jax.numpy as jnp
from jax import lax
from jax.experimental import pallas as pl
from jax.experimental.pallas import tpu as pltpu
from jax.experimental.pallas import tpu_sc as plsc

D_MODEL = 64
MAX_REL = 128
SEQ_LEN = 2048
N_TABLE = 2 * MAX_REL + 1

NUM_CORES = 2
NUM_SUBCORES = 16
NW = NUM_CORES * NUM_SUBCORES
ROWS_PER_W = SEQ_LEN // NW
COL_HALF = SEQ_LEN // 2
WIN = COL_HALF + ROWS_PER_W
IDX_CHUNK = 128
N_CHUNKS = (WIN + IDX_CHUNK - 1) // IDX_CHUNK
WIN_PAD = N_CHUNKS * IDX_CHUNK
LANES = 16
WRITE_BATCH = 8


def _sc_body(table_hbm, out_hbm, win_ref, idx_ref, sem_g, sem_w):
    c = lax.axis_index("c")
    s = lax.axis_index("s")
    wid = c * NUM_SUBCORES + s
    row0 = wid * ROWS_PER_W

    for h in range(2):
        b0 = h * COL_HALF - row0 + (MAX_REL - ROWS_PER_W + 1)

        def gen_idx(t, carry):
            vals = b0 + t * LANES + lax.iota(jnp.int32, LANES)
            idx_ref[pl.ds(t * LANES, LANES)] = jnp.clip(vals, 0, N_TABLE - 1)
            return carry

        lax.fori_loop(0, WIN_PAD // LANES, gen_idx, 0)

        gathers = [
            pltpu.async_copy(
                table_hbm.at[idx_ref.at[pl.ds(k * IDX_CHUNK, IDX_CHUNK)]],
                win_ref.at[pl.ds(k * IDX_CHUNK, IDX_CHUNK)],
                sem_g,
            )
            for k in range(N_CHUNKS)
        ]
        for g in gathers:
            g.wait()

        def write_batch(t, carry):
            handles = []
            for u in range(WRITE_BATCH):
                r = t * WRITE_BATCH + u
                handles.append(
                    pltpu.async_copy(
                        win_ref.at[pl.ds(ROWS_PER_W - 1 - r, COL_HALF)],
                        out_hbm.at[row0 + r, pl.ds(h * COL_HALF, COL_HALF)],
                        sem_w,
                    )
                )
            for hd in handles:
                hd.wait()
            return carry

        lax.fori_loop(0, ROWS_PER_W // WRITE_BATCH, write_batch, 0)


def kernel(seq_len, table):
    del seq_len
    mesh = plsc.VectorSubcoreMesh(
        core_axis_name="c", subcore_axis_name="s", num_cores=NUM_CORES
    )
    run = pl.kernel(
        _sc_body,
        out_type=jax.ShapeDtypeStruct((SEQ_LEN, SEQ_LEN, D_MODEL), jnp.float32),
        mesh=mesh,
        scratch_types=[
            pltpu.VMEM((WIN_PAD, D_MODEL), jnp.float32),
            pltpu.VMEM((WIN_PAD,), jnp.int32),
            pltpu.SemaphoreType.DMA,
            pltpu.SemaphoreType.DMA,
        ],
        compiler_params=pltpu.CompilerParams(use_tc_tiling_on_sc=False),
    )
    return run(table)
